# baseline (speedup 1.0000x reference)
import functools

import jax
import jax.numpy as jnp
from jax import lax
from jax.experimental import pallas as pl
from jax.experimental.pallas import tpu as pltpu
from jax.experimental.pallas import tpu_sc as plsc

NC = 2
NS = 16
NW = NC * NS
CHUNK = 128
DEGW = 16

_f32 = jnp.float32


def _sc_mesh():
    return plsc.VectorSubcoreMesh(
        core_axis_name="c", subcore_axis_name="s", num_cores=NC, num_subcores=NS
    )


def _make_deg_kernel(n_nodes: int, n_chunks: int, chunk: int):
    rpt = n_nodes // NS
    assert n_chunks >= 3

    @functools.partial(
        pl.kernel,
        out_type=jax.ShapeDtypeStruct((NC, n_nodes, DEGW), _f32),
        mesh=_sc_mesh(),
        compiler_params=pltpu.CompilerParams(use_tc_tiling_on_sc=False),
        scratch_types=[
            pltpu.VMEM((n_chunks, chunk), jnp.int32),
            pltpu.VMEM((chunk, DEGW), _f32),
            pltpu.VMEM_SHARED((n_nodes, DEGW), _f32),
            pltpu.SemaphoreType.DMA,
            pltpu.SemaphoreType.DMA,
        ],
    )
    def deg_kernel(dst_hbm, ones_hbm, zeros_hbm, out_hbm, dst_v, ones_v, acc,
                   ssem0, ssem1):
        c = lax.axis_index("c")
        s = lax.axis_index("s")
        w = c * NS + s
        pltpu.sync_copy(dst_hbm.at[w], dst_v)
        pltpu.sync_copy(ones_hbm, ones_v)
        pltpu.sync_copy(zeros_hbm, acc.at[pl.ds(s * rpt, rpt)])
        plsc.subcore_barrier()

        ssem = (ssem0, ssem1)
        pltpu.async_copy(ones_v, acc.at[dst_v.at[0]], ssem0, add=True)
        pltpu.async_copy(ones_v, acc.at[dst_v.at[1]], ssem1, add=True)

        def one(j, b):
            pltpu.make_async_copy(ones_v, acc.at[dst_v.at[j]], ssem[b]).wait()
            pltpu.async_copy(ones_v, acc.at[dst_v.at[j]], ssem[b], add=True)

        def body(i, carry):
            one(2 * i, 0)
            one(2 * i + 1, 1)
            return carry

        lax.fori_loop(1, n_chunks // 2, body, 0)
        if n_chunks % 2:
            one(n_chunks - 1, 0)
        pltpu.make_async_copy(ones_v, acc.at[dst_v.at[0]], ssem0).wait()
        pltpu.make_async_copy(ones_v, acc.at[dst_v.at[0]], ssem1).wait()
        plsc.subcore_barrier()
        pltpu.sync_copy(
            acc.at[pl.ds(s * rpt, rpt)], out_hbm.at[c, pl.ds(s * rpt, rpt)]
        )

    return deg_kernel


def _make_seg_sum_kernel(n_nodes: int, epw: int, d: int, chunk: int, nbuf: int):
    rpt = n_nodes // NS
    assert epw % chunk == 0 and chunk % 8 == 0 and epw % 8 == 0
    n_chunks = epw // chunk
    assert n_chunks >= 2 * nbuf
    rem = (n_chunks - nbuf) % nbuf
    params = None if d % 128 == 0 else pltpu.CompilerParams(use_tc_tiling_on_sc=False)

    scratch = (
        [pltpu.VMEM((chunk,), jnp.int32) for _ in range(2 * nbuf)]
        + [pltpu.VMEM((chunk, d), _f32) for _ in range(nbuf)]
        + [pltpu.VMEM_SHARED((n_nodes, d), _f32)]
        + [pltpu.SemaphoreType.DMA for _ in range(4 * nbuf)]
    )

    @functools.partial(
        pl.kernel,
        out_type=jax.ShapeDtypeStruct((NC, n_nodes, d), _f32),
        mesh=_sc_mesh(),
        compiler_params=params,
        scratch_types=scratch,
    )
    def seg_sum(table_hbm, src_hbm, dst_hbm, junk_hbm, zeros_hbm, out_hbm, *scr):
        sidx = scr[0:nbuf]
        didx = scr[nbuf:2 * nbuf]
        bufs = scr[2 * nbuf:3 * nbuf]
        acc = scr[3 * nbuf]
        gsem = scr[3 * nbuf + 1:4 * nbuf + 1]
        ssem = scr[4 * nbuf + 1:5 * nbuf + 1]
        isem = scr[5 * nbuf + 1:6 * nbuf + 1]
        dsem = scr[6 * nbuf + 1:7 * nbuf + 1]
        c = lax.axis_index("c")
        s = lax.axis_index("s")
        base = (c * NS + s) * epw

        def schunk(j):
            return src_hbm.at[pl.ds(base + j * chunk, chunk)]

        def dchunk(j):
            return dst_hbm.at[pl.ds(base + j * chunk, chunk)]

        pltpu.sync_copy(zeros_hbm, acc.at[pl.ds(s * rpt, rpt)])
        plsc.subcore_barrier()

        for q in range(nbuf):
            pltpu.async_copy(schunk(q), sidx[q], isem[q])
        for q in range(nbuf - 1):
            pltpu.async_copy(dchunk(q), didx[q], dsem[q])
        pltpu.sync_copy(junk_hbm, didx[nbuf - 1])
        pltpu.async_copy(bufs[nbuf - 1], acc.at[didx[nbuf - 1]], ssem[nbuf - 1], add=True)
        for q in range(nbuf - 1):
            pltpu.make_async_copy(schunk(q), sidx[q], isem[q]).wait()
            pltpu.async_copy(table_hbm.at[sidx[q]], bufs[q], gsem[q])

        def one(j, b, pre_src, pre_gather):
            pb = (b + nbuf - 1) % nbuf
            pltpu.make_async_copy(table_hbm.at[sidx[b]], bufs[b], gsem[b]).wait()
            if pre_src:
                pltpu.async_copy(schunk(j + nbuf), sidx[b], isem[b])
            pltpu.make_async_copy(bufs[pb], acc.at[didx[pb]], ssem[pb]).wait()
            if pre_gather:
                pltpu.async_copy(dchunk(j + nbuf - 1), didx[pb], dsem[pb])
                pltpu.make_async_copy(schunk(j + nbuf - 1), sidx[pb], isem[pb]).wait()
                pltpu.async_copy(table_hbm.at[sidx[pb]], bufs[pb], gsem[pb])
            pltpu.make_async_copy(dchunk(j), didx[b], dsem[b]).wait()
            pltpu.async_copy(bufs[b], acc.at[didx[b]], ssem[b], add=True)

        def steady(i, carry):
            for q in range(nbuf):
                one(i * nbuf + q, q, True, True)
            return carry

        nsteady = (n_chunks - nbuf - rem) // nbuf
        lax.fori_loop(0, nsteady, steady, 0)
        for r in range(rem):
            one(nsteady * nbuf + r, r, True, True)
        jt = n_chunks - nbuf
        one(jt, jt % nbuf, False, True)
        for q in range(1, nbuf):
            one(jt + q, (jt + q) % nbuf, False, False)
        pltpu.make_async_copy(
            bufs[(n_chunks - 1) % nbuf],
            acc.at[didx[(n_chunks - 1) % nbuf]],
            ssem[(n_chunks - 1) % nbuf],
        ).wait()

        plsc.subcore_barrier()
        pltpu.sync_copy(
            acc.at[pl.ds(s * rpt, rpt)], out_hbm.at[c, pl.ds(s * rpt, rpt)]
        )

    return seg_sum


def _tc_stage_a(x, w1, degp, bn: int):
    n, d_in = x.shape
    d_h = w1.shape[1]

    def body(x_ref, w1_ref, degp_ref, table_ref, dinv_ref):
        deg = (
            jnp.sum(degp_ref[0], axis=-1) + jnp.sum(degp_ref[1], axis=-1)
        ) * (1.0 / DEGW) + 1.0
        dinv = lax.rsqrt(deg)
        p = jnp.dot(x_ref[...], w1_ref[...], preferred_element_type=_f32)
        table_ref[...] = p * dinv[:, None]
        dinv_ref[...] = dinv[:, None]

    return pl.pallas_call(
        body,
        grid=(n // bn,),
        in_specs=[
            pl.BlockSpec((bn, d_in), lambda i: (i, 0)),
            pl.BlockSpec((d_in, d_h), lambda i: (0, 0)),
            pl.BlockSpec((NC, bn, DEGW), lambda i: (0, i, 0)),
        ],
        out_specs=[
            pl.BlockSpec((bn, d_h), lambda i: (i, 0)),
            pl.BlockSpec((bn, 1), lambda i: (i, 0)),
        ],
        out_shape=[
            jax.ShapeDtypeStruct((n, d_h), _f32),
            jax.ShapeDtypeStruct((n, 1), _f32),
        ],
    )(x, w1, degp)


def _tc_stage_b(aggp, table1, dinv, b1, w2p, bn: int):
    n, d_h = table1.shape
    d2 = w2p.shape[1]

    def body(aggp_ref, t1_ref, dinv_ref, b1_ref, w2_ref, out_ref):
        dv = dinv_ref[...]
        h = dv * (aggp_ref[0] + aggp_ref[1] + t1_ref[...]) + b1_ref[...]
        h = jnp.maximum(h, 0.0)
        q = jnp.dot(h, w2_ref[...], preferred_element_type=_f32)
        out_ref[...] = q * dv

    return pl.pallas_call(
        body,
        grid=(n // bn,),
        in_specs=[
            pl.BlockSpec((NC, bn, d_h), lambda i: (0, i, 0)),
            pl.BlockSpec((bn, d_h), lambda i: (i, 0)),
            pl.BlockSpec((bn, 1), lambda i: (i, 0)),
            pl.BlockSpec((1, d_h), lambda i: (0, 0)),
            pl.BlockSpec((d_h, d2), lambda i: (0, 0)),
        ],
        out_specs=pl.BlockSpec((bn, d2), lambda i: (i, 0)),
        out_shape=jax.ShapeDtypeStruct((n, d2), _f32),
    )(aggp, table1, dinv, b1, w2p)


def _tc_stage_c(aggp, table2, dinv, b2p, d_out: int, bn: int):
    n, d2 = table2.shape

    def body(aggp_ref, t2_ref, dinv_ref, b2_ref, out_ref):
        dv = dinv_ref[...]
        o = dv * (aggp_ref[0] + aggp_ref[1] + t2_ref[...]) + b2_ref[...]
        col = lax.broadcasted_iota(jnp.int32, o.shape, 1)
        valid = col < d_out
        neg = jnp.full_like(o, -jnp.inf)
        logits = jnp.where(valid, o, neg)
        m = jnp.max(logits, axis=-1, keepdims=True)
        lse = jnp.log(jnp.sum(jnp.exp(logits - m), axis=-1, keepdims=True)) + m
        out_ref[...] = (o - lse)[:, :d_out]

    return pl.pallas_call(
        body,
        grid=(n // bn,),
        in_specs=[
            pl.BlockSpec((NC, bn, d2), lambda i: (0, i, 0)),
            pl.BlockSpec((bn, d2), lambda i: (i, 0)),
            pl.BlockSpec((bn, 1), lambda i: (i, 0)),
            pl.BlockSpec((1, d2), lambda i: (0, 0)),
        ],
        out_specs=pl.BlockSpec((bn, d_out), lambda i: (i, 0)),
        out_shape=jax.ShapeDtypeStruct((n, d_out), _f32),
    )(aggp, table2, dinv, b2p)


def kernel(x, edge_index, W1, b1, W2, b2):
    n, d_in = x.shape
    e = edge_index.shape[1]
    d_h = W1.shape[1]
    d_out = W2.shape[1]
    d2 = 48
    npad = -(-n // (NS * 8)) * (NS * 8) + (NS * 8 if n % (NS * 8) == 0 else 0)

    assert e % NW == 0
    epw = e // NW
    ck, nb = 80, 4
    assert epw % ck == 0
    nck = epw // ck

    src_flat = edge_index[0]
    dst_flat = edge_index[1]
    dst_deg = edge_index[1].reshape(NW, nck, ck)

    rpt = npad // NS
    ones_deg = jnp.ones((ck, DEGW), _f32)
    zeros_deg = jnp.zeros((rpt, DEGW), _f32)
    zeros_h = jnp.zeros((rpt, d_h), _f32)
    zeros_2 = jnp.zeros((rpt, d2), _f32)
    w2p = jnp.pad(W2, ((0, 0), (0, d2 - d_out)))
    b1r = b1.reshape(1, d_h)
    b2p = jnp.pad(b2, (0, d2 - d_out)).reshape(1, d2)
    junk = (n + jnp.arange(ck, dtype=jnp.int32) % (npad - n)).astype(jnp.int32)

    degp = _make_deg_kernel(npad, nck, ck)(dst_deg, ones_deg, zeros_deg)
    table1, dinv = _tc_stage_a(x, W1, degp, bn=2000)
    aggp1 = _make_seg_sum_kernel(npad, epw, d_h, ck, nb)(
        table1, src_flat, dst_flat, junk, zeros_h)
    table2 = _tc_stage_b(aggp1, table1, dinv, b1r, w2p, bn=2000)
    aggp2 = _make_seg_sum_kernel(npad, epw, d2, ck, nb)(
        table2, src_flat, dst_flat, junk, zeros_2)
    return _tc_stage_c(aggp2, table2, dinv, b2p, d_out, bn=2000)


# speedup vs baseline: 1.0314x; 1.0314x over previous
import functools

import jax
import jax.numpy as jnp
from jax import lax
from jax.experimental import pallas as pl
from jax.experimental.pallas import tpu as pltpu
from jax.experimental.pallas import tpu_sc as plsc

NC = 2
NS = 16
NW = NC * NS
CHUNK = 128
DEGW = 16

_f32 = jnp.float32


def _sc_mesh():
    return plsc.VectorSubcoreMesh(
        core_axis_name="c", subcore_axis_name="s", num_cores=NC, num_subcores=NS
    )


def _make_deg_kernel(n_nodes: int, n_chunks: int, chunk: int):
    rpt = n_nodes // NS
    assert n_chunks >= 3

    @functools.partial(
        pl.kernel,
        out_type=jax.ShapeDtypeStruct((NC, n_nodes, DEGW), _f32),
        mesh=_sc_mesh(),
        compiler_params=pltpu.CompilerParams(use_tc_tiling_on_sc=False),
        scratch_types=[
            pltpu.VMEM((n_chunks, chunk), jnp.int32),
            pltpu.VMEM((chunk, DEGW), _f32),
            pltpu.VMEM_SHARED((n_nodes, DEGW), _f32),
            pltpu.SemaphoreType.DMA,
            pltpu.SemaphoreType.DMA,
        ],
    )
    def deg_kernel(dst_hbm, ones_hbm, zeros_hbm, out_hbm, dst_v, ones_v, acc,
                   ssem0, ssem1):
        c = lax.axis_index("c")
        s = lax.axis_index("s")
        w = c * NS + s
        pltpu.sync_copy(dst_hbm.at[w], dst_v)
        pltpu.sync_copy(ones_hbm, ones_v)
        pltpu.sync_copy(zeros_hbm, acc.at[pl.ds(s * rpt, rpt)])
        plsc.subcore_barrier()

        ssem = (ssem0, ssem1)
        pltpu.async_copy(ones_v, acc.at[dst_v.at[0]], ssem0, add=True)
        pltpu.async_copy(ones_v, acc.at[dst_v.at[1]], ssem1, add=True)

        def one(j, b):
            pltpu.make_async_copy(ones_v, acc.at[dst_v.at[j]], ssem[b]).wait()
            pltpu.async_copy(ones_v, acc.at[dst_v.at[j]], ssem[b], add=True)

        def body(i, carry):
            one(2 * i, 0)
            one(2 * i + 1, 1)
            return carry

        lax.fori_loop(1, n_chunks // 2, body, 0)
        if n_chunks % 2:
            one(n_chunks - 1, 0)
        pltpu.make_async_copy(ones_v, acc.at[dst_v.at[0]], ssem0).wait()
        pltpu.make_async_copy(ones_v, acc.at[dst_v.at[0]], ssem1).wait()
        plsc.subcore_barrier()
        pltpu.sync_copy(
            acc.at[pl.ds(s * rpt, rpt)], out_hbm.at[c, pl.ds(s * rpt, rpt)]
        )

    return deg_kernel


def _make_seg_sum_kernel(n_nodes: int, epw: int, d: int, chunk: int, nbuf: int):
    rpt = n_nodes // NS
    assert epw % chunk == 0 and chunk % 8 == 0 and epw % 8 == 0
    n_chunks = epw // chunk
    assert n_chunks >= 2 * nbuf
    rem = (n_chunks - nbuf) % nbuf
    params = None if d % 128 == 0 else pltpu.CompilerParams(use_tc_tiling_on_sc=False)

    scratch = (
        [pltpu.VMEM((chunk,), jnp.int32) for _ in range(2 * nbuf)]
        + [pltpu.VMEM((chunk, d), _f32) for _ in range(nbuf)]
        + [pltpu.VMEM_SHARED((n_nodes, d), _f32)]
        + [pltpu.SemaphoreType.DMA for _ in range(4 * nbuf)]
    )

    @functools.partial(
        pl.kernel,
        out_type=jax.ShapeDtypeStruct((NC, n_nodes, d), _f32),
        mesh=_sc_mesh(),
        compiler_params=params,
        scratch_types=scratch,
    )
    def seg_sum(table_hbm, src_hbm, dst_hbm, junk_hbm, zeros_hbm, out_hbm, *scr):
        sidx = scr[0:nbuf]
        didx = scr[nbuf:2 * nbuf]
        bufs = scr[2 * nbuf:3 * nbuf]
        acc = scr[3 * nbuf]
        gsem = scr[3 * nbuf + 1:4 * nbuf + 1]
        ssem = scr[4 * nbuf + 1:5 * nbuf + 1]
        isem = scr[5 * nbuf + 1:6 * nbuf + 1]
        dsem = scr[6 * nbuf + 1:7 * nbuf + 1]
        c = lax.axis_index("c")
        s = lax.axis_index("s")
        base = (c * NS + s) * epw

        def schunk(j):
            return src_hbm.at[pl.ds(base + j * chunk, chunk)]

        def dchunk(j):
            return dst_hbm.at[pl.ds(base + j * chunk, chunk)]

        pltpu.sync_copy(zeros_hbm, acc.at[pl.ds(s * rpt, rpt)])
        plsc.subcore_barrier()

        for q in range(nbuf):
            pltpu.async_copy(schunk(q), sidx[q], isem[q])
        for q in range(nbuf - 1):
            pltpu.async_copy(dchunk(q), didx[q], dsem[q])
        pltpu.sync_copy(junk_hbm, didx[nbuf - 1])
        pltpu.async_copy(bufs[nbuf - 1], acc.at[didx[nbuf - 1]], ssem[nbuf - 1], add=True)
        for q in range(nbuf - 1):
            pltpu.make_async_copy(schunk(q), sidx[q], isem[q]).wait()
            pltpu.async_copy(table_hbm.at[sidx[q]], bufs[q], gsem[q])

        def one(j, b, pre_src, pre_gather):
            pb = (b + nbuf - 1) % nbuf
            pltpu.make_async_copy(table_hbm.at[sidx[b]], bufs[b], gsem[b]).wait()
            if pre_src:
                pltpu.async_copy(schunk(j + nbuf), sidx[b], isem[b])
            pltpu.make_async_copy(bufs[pb], acc.at[didx[pb]], ssem[pb]).wait()
            if pre_gather:
                pltpu.async_copy(dchunk(j + nbuf - 1), didx[pb], dsem[pb])
                pltpu.make_async_copy(schunk(j + nbuf - 1), sidx[pb], isem[pb]).wait()
                pltpu.async_copy(table_hbm.at[sidx[pb]], bufs[pb], gsem[pb])
            pltpu.make_async_copy(dchunk(j), didx[b], dsem[b]).wait()
            pltpu.async_copy(bufs[b], acc.at[didx[b]], ssem[b], add=True)

        def steady(i, carry):
            for q in range(nbuf):
                one(i * nbuf + q, q, True, True)
            return carry

        nsteady = (n_chunks - nbuf - rem) // nbuf
        lax.fori_loop(0, nsteady, steady, 0)
        for r in range(rem):
            one(nsteady * nbuf + r, r, True, True)
        jt = n_chunks - nbuf
        one(jt, jt % nbuf, False, True)
        for q in range(1, nbuf):
            one(jt + q, (jt + q) % nbuf, False, False)
        pltpu.make_async_copy(
            bufs[(n_chunks - 1) % nbuf],
            acc.at[didx[(n_chunks - 1) % nbuf]],
            ssem[(n_chunks - 1) % nbuf],
        ).wait()

        plsc.subcore_barrier()
        pltpu.sync_copy(
            acc.at[pl.ds(s * rpt, rpt)], out_hbm.at[c, pl.ds(s * rpt, rpt)]
        )

    return seg_sum


def _tc_stage_a(x, w1, degp, bn: int):
    n, d_in = x.shape
    d_h = w1.shape[1]

    def body(x_ref, w1_ref, degp_ref, table_ref, dinv_ref):
        deg = (
            jnp.sum(degp_ref[0], axis=-1) + jnp.sum(degp_ref[1], axis=-1)
        ) * (1.0 / DEGW) + 1.0
        dinv = lax.rsqrt(deg)
        p = jnp.dot(x_ref[...], w1_ref[...], preferred_element_type=_f32)
        table_ref[...] = p * dinv[:, None]
        dinv_ref[...] = dinv[:, None]

    return pl.pallas_call(
        body,
        grid=(n // bn,),
        in_specs=[
            pl.BlockSpec((bn, d_in), lambda i: (i, 0)),
            pl.BlockSpec((d_in, d_h), lambda i: (0, 0)),
            pl.BlockSpec((NC, bn, DEGW), lambda i: (0, i, 0)),
        ],
        out_specs=[
            pl.BlockSpec((bn, d_h), lambda i: (i, 0)),
            pl.BlockSpec((bn, 1), lambda i: (i, 0)),
        ],
        out_shape=[
            jax.ShapeDtypeStruct((n, d_h), _f32),
            jax.ShapeDtypeStruct((n, 1), _f32),
        ],
    )(x, w1, degp)


def _tc_stage_b(aggp, table1, dinv, b1, w2p, bn: int):
    n, d_h = table1.shape
    d2 = w2p.shape[1]

    def body(aggp_ref, t1_ref, dinv_ref, b1_ref, w2_ref, out_ref):
        dv = dinv_ref[...]
        h = dv * (aggp_ref[0] + aggp_ref[1] + t1_ref[...]) + b1_ref[...]
        h = jnp.maximum(h, 0.0)
        q = jnp.dot(h, w2_ref[...], preferred_element_type=_f32)
        out_ref[...] = q * dv

    return pl.pallas_call(
        body,
        grid=(n // bn,),
        in_specs=[
            pl.BlockSpec((NC, bn, d_h), lambda i: (0, i, 0)),
            pl.BlockSpec((bn, d_h), lambda i: (i, 0)),
            pl.BlockSpec((bn, 1), lambda i: (i, 0)),
            pl.BlockSpec((1, d_h), lambda i: (0, 0)),
            pl.BlockSpec((d_h, d2), lambda i: (0, 0)),
        ],
        out_specs=pl.BlockSpec((bn, d2), lambda i: (i, 0)),
        out_shape=jax.ShapeDtypeStruct((n, d2), _f32),
    )(aggp, table1, dinv, b1, w2p)


def _tc_stage_c(aggp, table2, dinv, b2p, d_out: int, bn: int):
    n, d2 = table2.shape

    def body(aggp_ref, t2_ref, dinv_ref, b2_ref, out_ref):
        dv = dinv_ref[...]
        o = dv * (aggp_ref[0] + aggp_ref[1] + t2_ref[...]) + b2_ref[...]
        col = lax.broadcasted_iota(jnp.int32, o.shape, 1)
        valid = col < d_out
        neg = jnp.full_like(o, -jnp.inf)
        logits = jnp.where(valid, o, neg)
        m = jnp.max(logits, axis=-1, keepdims=True)
        lse = jnp.log(jnp.sum(jnp.exp(logits - m), axis=-1, keepdims=True)) + m
        out_ref[...] = (o - lse)[:, :d_out]

    return pl.pallas_call(
        body,
        grid=(n // bn,),
        in_specs=[
            pl.BlockSpec((NC, bn, d2), lambda i: (0, i, 0)),
            pl.BlockSpec((bn, d2), lambda i: (i, 0)),
            pl.BlockSpec((bn, 1), lambda i: (i, 0)),
            pl.BlockSpec((1, d2), lambda i: (0, 0)),
        ],
        out_specs=pl.BlockSpec((bn, d_out), lambda i: (i, 0)),
        out_shape=jax.ShapeDtypeStruct((n, d_out), _f32),
    )(aggp, table2, dinv, b2p)


def kernel(x, edge_index, W1, b1, W2, b2):
    n, d_in = x.shape
    e = edge_index.shape[1]
    d_h = W1.shape[1]
    d_out = W2.shape[1]
    d2 = 48
    npad = -(-n // (NS * 8)) * (NS * 8) + (NS * 8 if n % (NS * 8) == 0 else 0)

    assert e % NW == 0
    epw = e // NW
    ck, nb = 80, 4
    assert epw % ck == 0
    nck = epw // ck

    src_flat = edge_index[0]
    dst_flat = edge_index[1]
    dst_deg = edge_index[1].reshape(NW, nck, ck)

    rpt = npad // NS
    ones_deg = jnp.ones((ck, DEGW), _f32)
    zeros_deg = jnp.zeros((rpt, DEGW), _f32)
    zeros_h = jnp.zeros((rpt, d_h), _f32)
    zeros_2 = jnp.zeros((rpt, d2), _f32)
    w2p = jnp.pad(W2, ((0, 0), (0, d2 - d_out)))
    b1r = b1.reshape(1, d_h)
    b2p = jnp.pad(b2, (0, d2 - d_out)).reshape(1, d2)
    junk = (n + jnp.arange(ck, dtype=jnp.int32) % (npad - n)).astype(jnp.int32)

    degp = _make_deg_kernel(npad, nck, ck)(dst_deg, ones_deg, zeros_deg)
    table1, dinv = _tc_stage_a(x, W1, degp, bn=2000)
    aggp1 = _make_seg_sum_kernel(npad, epw, d_h, ck, nb)(
        table1, src_flat, dst_flat, junk, zeros_h)
    table2 = _tc_stage_b(aggp1, table1, dinv, b1r, w2p, bn=2000)
    aggp2 = _make_seg_sum_kernel(npad, epw, d2, ck, 6)(
        table2, src_flat, dst_flat, junk, zeros_2)
    return _tc_stage_c(aggp2, table2, dinv, b2p, d_out, bn=2000)
